# baseline (speedup 1.0000x reference)
import jax
import jax.numpy as jnp
from jax.experimental import pallas as pl
from jax.experimental.pallas import tpu as pltpu

D = 768
FF = 3072
E = 16

F_BLK_A = 1536
F_BLK_B = 1536


def _shared_kernel(xf_ref, h_ref, sw1_ref, sw3_ref, sw2_ref, xps_ref):
    step = pl.program_id(0)

    @pl.when(step == 0)
    def _():
        xps_ref[:] = xf_ref[:]

    hb = h_ref[:]
    g = jax.lax.dot_general(hb, sw1_ref[:].astype(jnp.bfloat16),
                            (((1,), (1,)), ((), ())),
                            preferred_element_type=jnp.float32)
    u = jax.lax.dot_general(hb, sw3_ref[:].astype(jnp.bfloat16),
                            (((1,), (1,)), ((), ())),
                            preferred_element_type=jnp.float32)
    act = (jax.nn.silu(g) * u).astype(jnp.bfloat16)
    xps_ref[:] += jax.lax.dot_general(act, sw2_ref[:].astype(jnp.bfloat16),
                                      (((1,), (1,)), ((), ())),
                                      preferred_element_type=jnp.float32)


def _moe_kernel(h_ref, routing_ref, xps_ref, w1_ref, w3_ref, w2_ref, out_ref):
    e = pl.program_id(0)
    f = pl.program_id(1)
    T = h_ref.shape[0]

    @pl.when((e == 0) & (f == 0))
    def _():
        out_ref[:] = xps_ref[:]

    hb = h_ref[:]
    g = jax.lax.dot_general(hb, w1_ref[0].astype(jnp.bfloat16),
                            (((1,), (1,)), ((), ())),
                            preferred_element_type=jnp.float32)
    u = jax.lax.dot_general(hb, w3_ref[0].astype(jnp.bfloat16),
                            (((1,), (1,)), ((), ())),
                            preferred_element_type=jnp.float32)
    act = jax.nn.silu(g) * u
    lane = jax.lax.broadcasted_iota(jnp.int32, (T, E), 1)
    col = jnp.sum(jnp.where(lane == e, routing_ref[:], 0.0), axis=1,
                  keepdims=True)
    act = (act * col).astype(jnp.bfloat16)
    out_ref[:] += jax.lax.dot_general(act, w2_ref[0].astype(jnp.bfloat16),
                                      (((1,), (1,)), ((), ())),
                                      preferred_element_type=jnp.float32)


def kernel(x, time_c, ada_w, ada_b, gate_w, w1, w3, w2, sw1, sw3, sw2):
    B, L, Dm = x.shape
    T = B * L

    cond = jax.nn.silu(time_c)
    ss = cond @ ada_w.T + ada_b
    shift, scale = jnp.split(ss, 2, axis=-1)
    m = jnp.mean(x, -1, keepdims=True)
    v = jnp.var(x, -1, keepdims=True)
    xn = (x - m) * jax.lax.rsqrt(v + 1e-5) * (1.0 + scale[:, None, :]) \
        + shift[:, None, :]
    h = xn.reshape(-1, Dm)
    logits = h @ gate_w.T
    probs = jax.nn.softmax(logits, axis=-1)
    topv, topi = jax.lax.top_k(probs, 2)
    topv = topv / jnp.sum(topv, axis=-1, keepdims=True)
    routing = jnp.zeros((T, E), x.dtype).at[
        jnp.arange(T)[:, None], topi].set(topv)

    xf = x.reshape(T, Dm)
    hb = h.astype(jnp.bfloat16)

    nfa = FF // F_BLK_A
    xps = pl.pallas_call(
        _shared_kernel,
        grid=(nfa,),
        in_specs=[
            pl.BlockSpec((T, Dm), lambda f: (0, 0)),
            pl.BlockSpec((T, Dm), lambda f: (0, 0)),
            pl.BlockSpec((F_BLK_A, Dm), lambda f: (f, 0)),
            pl.BlockSpec((F_BLK_A, Dm), lambda f: (f, 0)),
            pl.BlockSpec((Dm, F_BLK_A), lambda f: (0, f)),
        ],
        out_specs=pl.BlockSpec((T, Dm), lambda f: (0, 0)),
        out_shape=jax.ShapeDtypeStruct((T, Dm), jnp.float32),
        compiler_params=pltpu.CompilerParams(
            dimension_semantics=("arbitrary",)),
    )(xf, hb, sw1, sw3, sw2)

    nfb = FF // F_BLK_B
    y = pl.pallas_call(
        _moe_kernel,
        grid=(E, nfb),
        in_specs=[
            pl.BlockSpec((T, Dm), lambda e, f: (0, 0)),
            pl.BlockSpec((T, E), lambda e, f: (0, 0)),
            pl.BlockSpec((T, Dm), lambda e, f: (0, 0)),
            pl.BlockSpec((1, F_BLK_B, Dm), lambda e, f: (e, f, 0)),
            pl.BlockSpec((1, F_BLK_B, Dm), lambda e, f: (e, f, 0)),
            pl.BlockSpec((1, Dm, F_BLK_B), lambda e, f: (e, 0, f)),
        ],
        out_specs=pl.BlockSpec((T, Dm), lambda e, f: (0, 0)),
        out_shape=jax.ShapeDtypeStruct((T, Dm), jnp.float32),
        compiler_params=pltpu.CompilerParams(
            dimension_semantics=("arbitrary", "arbitrary")),
    )(hb, routing, xps, w1, w3, w2)

    return y.reshape(B, L, Dm)
